# baseline (speedup 1.0000x reference)
import functools

import jax
import jax.numpy as jnp
from jax import lax
from jax.experimental import pallas as pl
from jax.experimental.pallas import tpu as pltpu
from jax.experimental.pallas import tpu_sc as plsc

B = 128
P = 10
K = 5
LP = 5
D = 1000
PE_W = (K * LP + 1) * D
PR_W = K * LP * D
KR_W = K * D
NW = 32
RPW = B // NW


def _l2n(v):
    return v * lax.rsqrt(jnp.maximum(jnp.sum(v * v, axis=1, keepdims=True), 1e-12))


def _tc_body(x_ref, pf_ref, pk_ref,
             idx_ref, pn_ref, xn_ref, sim_ref, rs_ref, prow_ref, krow_ref):
    x = x_ref[...]
    pk = pk_ref[...]
    pf = pf_ref[...]

    pn = _l2n(pk)
    xn = _l2n(x)
    sim = lax.dot_general(xn, pn, (((1,), (1,)), ((), ())))

    colj = lax.broadcasted_iota(jnp.int32, (1, P), 1)
    rank = jnp.zeros((B, P), jnp.int32)
    for jp in range(P):
        sj = sim[:, jp:jp + 1]
        gt = (sj > sim).astype(jnp.int32)
        eq = (sj == sim).astype(jnp.int32) * (colj > jp).astype(jnp.int32)
        rank = rank + gt + eq
    in_top = (rank < K).astype(jnp.int32)
    counts = jnp.sum(in_top, axis=0, keepdims=True)

    crank = jnp.zeros((1, P), jnp.int32)
    for jp in range(P):
        cj = counts[:, jp:jp + 1]
        gt = (cj > counts).astype(jnp.int32)
        eq = (cj == counts).astype(jnp.int32) * (colj > jp).astype(jnp.int32)
        crank = crank + gt + eq
    sel = crank < K
    self32 = sel.astype(jnp.float32)

    r_io = lax.broadcasted_iota(jnp.int32, (P, P), 0)
    c_io = lax.broadcasted_iota(jnp.int32, (P, P), 1)
    strict_lt = (r_io < c_io).astype(jnp.float32)
    pos = lax.dot_general(self32, strict_lt, (((1,), (0,)), ((), ())))

    s_io = lax.broadcasted_iota(jnp.int32, (K, P), 0).astype(jnp.float32)
    oh = ((s_io == pos) & sel).astype(jnp.float32)

    coljf = colj.astype(jnp.float32)
    major_f = lax.dot_general(coljf, oh, (((1,), (1,)), ((), ())))

    sel_key = lax.dot_general(oh, pn, (((1,), (0,)), ((), ())))
    sel_pr = lax.dot_general(oh, pf, (((1,), (0,)), ((), ())))

    krow_ref[...] = jnp.concatenate(
        [sel_key[s:s + 1, :] for s in range(K)], axis=1)
    prow_ref[...] = jnp.concatenate(
        [sel_pr[s:s + 1, :] for s in range(K)], axis=1)

    idx_ref[...] = jnp.broadcast_to(major_f.astype(jnp.int32), (B, K))
    pn_ref[...] = pn
    xn_ref[...] = xn
    sim_ref[...] = sim

    ksum = jnp.sum(sel_key, axis=0, keepdims=True)
    xnsum = jnp.sum(xn, axis=0, keepdims=True)
    rs_ref[...] = (jnp.sum(ksum * xnsum) / B).reshape(1, 1)


_sc_mesh = plsc.VectorSubcoreMesh(core_axis_name="c", subcore_axis_name="s")


@functools.partial(
    pl.kernel,
    mesh=_sc_mesh,
    out_type=[
        jax.ShapeDtypeStruct((B * PE_W,), jnp.float32),
        jax.ShapeDtypeStruct((B * KR_W,), jnp.float32),
    ],
    scratch_types=[
        pltpu.VMEM((PR_W,), jnp.float32),
        pltpu.VMEM((KR_W,), jnp.float32),
        pltpu.VMEM((RPW * D,), jnp.float32),
        pltpu.SemaphoreType.DMA,
    ],
)
def _sc_broadcast(prow_hbm, krow_hbm, x_hbm, pe_hbm, bkn_hbm,
                  prow_v, krow_v, xr_v, sem):
    cid = lax.axis_index("c")
    sid = lax.axis_index("s")
    wid = sid * 2 + cid
    base = wid * RPW

    ld0 = pltpu.async_copy(prow_hbm, prow_v, sem)
    ld1 = pltpu.async_copy(krow_hbm, krow_v, sem)
    ld2 = pltpu.async_copy(x_hbm.at[pl.ds(base * D, RPW * D)], xr_v, sem)
    ld0.wait()
    ld1.wait()
    ld2.wait()

    cps = []
    for r in range(RPW):
        row = base + r
        cps.append(pltpu.async_copy(
            prow_v, pe_hbm.at[pl.ds(row * PE_W, PR_W)], sem))
        cps.append(pltpu.async_copy(
            xr_v.at[pl.ds(r * D, D)], pe_hbm.at[pl.ds(row * PE_W + PR_W, D)], sem))
        cps.append(pltpu.async_copy(
            krow_v, bkn_hbm.at[pl.ds(row * KR_W, KR_W)], sem))
    for cp in cps:
        cp.wait()


def kernel(x, prompt, prompt_key):
    pf = prompt.reshape(P, LP * D)
    idx_b, pn, xn, sim, rs, prow, krow = pl.pallas_call(
        _tc_body,
        out_shape=[
            jax.ShapeDtypeStruct((B, K), jnp.int32),
            jax.ShapeDtypeStruct((P, D), jnp.float32),
            jax.ShapeDtypeStruct((B, D), jnp.float32),
            jax.ShapeDtypeStruct((B, P), jnp.float32),
            jax.ShapeDtypeStruct((1, 1), jnp.float32),
            jax.ShapeDtypeStruct((1, PR_W), jnp.float32),
            jax.ShapeDtypeStruct((1, KR_W), jnp.float32),
        ],
    )(x, pf, prompt_key)

    pe, bkn = _sc_broadcast(prow.reshape(PR_W), krow.reshape(KR_W),
                            x.reshape(B * D))
    return (idx_b, pn, xn, sim, bkn.reshape(B, K, D), rs[0, 0],
            pe.reshape(B, PE_W))


# speedup vs baseline: 2.0387x; 2.0387x over previous
import jax
import jax.numpy as jnp
from jax import lax
from jax.experimental import pallas as pl
from jax.experimental.pallas import tpu as pltpu

B = 128
P = 10
K = 5
LP = 5
D = 1000
BLK = 64
GRID = B // BLK
PE_W = (K * LP + 1) * D


def _l2n(v):
    return v * lax.rsqrt(jnp.maximum(jnp.sum(v * v, axis=1, keepdims=True), 1e-12))


def _body(x_ref, pf_ref, pk_ref,
          idx_ref, pn_ref, xn_ref, sim_ref, bkn_ref, rs_ref, pe_ref,
          prow_ref, krow_ref, major_ref, pns_ref):
    i = pl.program_id(0)

    @pl.when(i == 0)
    def _():
        x = x_ref[...]
        pk = pk_ref[...]
        pf = pf_ref[...]

        pn = _l2n(pk)
        xn = _l2n(x)
        sim = lax.dot_general(xn, pn, (((1,), (1,)), ((), ())))

        colj = lax.broadcasted_iota(jnp.int32, (1, P), 1)
        rank = jnp.zeros((B, P), jnp.int32)
        for jp in range(P):
            sj = sim[:, jp:jp + 1]
            gt = (sj > sim).astype(jnp.int32)
            eq = (sj == sim).astype(jnp.int32) * (colj > jp).astype(jnp.int32)
            rank = rank + gt + eq
        in_top = (rank < K).astype(jnp.int32)
        counts = jnp.sum(in_top, axis=0, keepdims=True)

        crank = jnp.zeros((1, P), jnp.int32)
        for jp in range(P):
            cj = counts[:, jp:jp + 1]
            gt = (cj > counts).astype(jnp.int32)
            eq = (cj == counts).astype(jnp.int32) * (colj > jp).astype(jnp.int32)
            crank = crank + gt + eq
        sel = crank < K
        self32 = sel.astype(jnp.float32)

        r_io = lax.broadcasted_iota(jnp.int32, (P, P), 0)
        c_io = lax.broadcasted_iota(jnp.int32, (P, P), 1)
        strict_lt = (r_io < c_io).astype(jnp.float32)
        pos = lax.dot_general(self32, strict_lt, (((1,), (0,)), ((), ())))

        s_io = lax.broadcasted_iota(jnp.int32, (K, P), 0).astype(jnp.float32)
        oh = ((s_io == pos) & sel).astype(jnp.float32)

        coljf = colj.astype(jnp.float32)
        major_f = lax.dot_general(coljf, oh, (((1,), (1,)), ((), ())))

        sel_key = lax.dot_general(oh, pn, (((1,), (0,)), ((), ())))
        sel_pr = lax.dot_general(oh, pf, (((1,), (0,)), ((), ())))

        krow_ref[...] = jnp.concatenate(
            [sel_key[s:s + 1, :] for s in range(K)], axis=1)
        prow_ref[...] = jnp.concatenate(
            [sel_pr[s:s + 1, :] for s in range(K)], axis=1)
        major_ref[...] = major_f.astype(jnp.int32)
        pns_ref[...] = pn
        pn_ref[...] = pn

        ksum = jnp.sum(sel_key, axis=0, keepdims=True)
        xnsum = jnp.sum(xn, axis=0, keepdims=True)
        rs_ref[...] = (jnp.sum(ksum * xnsum) / B).reshape(1, 1)

    x_blk = x_ref[pl.ds(i * BLK, BLK), :]
    xn_blk = _l2n(x_blk)
    pn = pns_ref[...]
    sim_blk = lax.dot_general(xn_blk, pn, (((1,), (1,)), ((), ())))

    idx_ref[...] = jnp.broadcast_to(major_ref[...], (BLK, K))
    xn_ref[...] = xn_blk
    sim_ref[...] = sim_blk
    bkn_ref[...] = jnp.broadcast_to(krow_ref[...], (BLK, K * D))
    pe_ref[...] = jnp.concatenate(
        [jnp.broadcast_to(prow_ref[...], (BLK, K * LP * D)), x_blk], axis=1)


def kernel(x, prompt, prompt_key):
    pf = prompt.reshape(P, LP * D)
    outs = pl.pallas_call(
        _body,
        grid=(GRID,),
        in_specs=[
            pl.BlockSpec((B, D), lambda i: (0, 0)),
            pl.BlockSpec((P, LP * D), lambda i: (0, 0)),
            pl.BlockSpec((P, D), lambda i: (0, 0)),
        ],
        out_specs=[
            pl.BlockSpec((BLK, K), lambda i: (i, 0)),
            pl.BlockSpec((P, D), lambda i: (0, 0)),
            pl.BlockSpec((BLK, D), lambda i: (i, 0)),
            pl.BlockSpec((BLK, P), lambda i: (i, 0)),
            pl.BlockSpec((BLK, K * D), lambda i: (i, 0)),
            pl.BlockSpec((1, 1), lambda i: (0, 0)),
            pl.BlockSpec((BLK, PE_W), lambda i: (i, 0)),
        ],
        out_shape=[
            jax.ShapeDtypeStruct((B, K), jnp.int32),
            jax.ShapeDtypeStruct((P, D), jnp.float32),
            jax.ShapeDtypeStruct((B, D), jnp.float32),
            jax.ShapeDtypeStruct((B, P), jnp.float32),
            jax.ShapeDtypeStruct((B, K * D), jnp.float32),
            jax.ShapeDtypeStruct((1, 1), jnp.float32),
            jax.ShapeDtypeStruct((B, PE_W), jnp.float32),
        ],
        scratch_shapes=[
            pltpu.VMEM((1, K * LP * D), jnp.float32),
            pltpu.VMEM((1, K * D), jnp.float32),
            pltpu.VMEM((1, K), jnp.int32),
            pltpu.VMEM((P, D), jnp.float32),
        ],
    )(x, pf, prompt_key)
    idx_b, pn, xn, sim, bkn, rs, pe = outs
    return (idx_b, pn, xn, sim, bkn.reshape(B, K, D), rs[0, 0], pe)
